# Initial kernel scaffold; baseline (speedup 1.0000x reference)
import jax
import jax.numpy as jnp
from jax.experimental import pallas as pl


def kernel(x, edge_index, batch, W1, b1, W2, b2, Wl, bl):
    raise NotImplementedError("write your pallas kernel here")


# speedup vs baseline: 18.6394x; 18.6394x over previous
import functools

import jax
import jax.numpy as jnp
from jax import lax
from jax.experimental import pallas as pl
from jax.experimental.pallas import tpu as pltpu
from jax.experimental.pallas import tpu_sc as plsc

N_NODES = 10000
N_EDGES = 320000
D = 128
N_GRAPHS = 64

N_WORKERS = 32
CHUNK = 80
N_CHUNKS = N_EDGES // (N_WORKERS * CHUNK)
N_PAD = 10240
ROWS_PER_TILE = N_PAD // 16
ROW_BLOCK = 2000
N_ROW_BLOCKS = N_NODES // ROW_BLOCK

_mesh = functools.partial(
    plsc.VectorSubcoreMesh, core_axis_name="c", subcore_axis_name="s")


def _sc_degree(dst_r, ones_v, zeros_v):
  @functools.partial(
      pl.kernel,
      out_type=jax.ShapeDtypeStruct((2, N_PAD, 16), jnp.float32),
      mesh=_mesh(),
      scratch_types=[
          pltpu.VMEM((N_CHUNKS, CHUNK), jnp.int32),
          pltpu.VMEM((CHUNK, 16), jnp.float32),
          pltpu.VMEM_SHARED((N_PAD, 16), jnp.float32),
      ],
  )
  def k(dst_hbm, ones_hbm, zeros_hbm, out_hbm, dst_v, ones_t, acc):
    c = lax.axis_index("c")
    s = lax.axis_index("s")
    wid = c * 16 + s
    pltpu.sync_copy(dst_hbm.at[wid], dst_v)
    pltpu.sync_copy(ones_hbm, ones_t)

    @pl.when(s == 0)
    def _():
      pltpu.sync_copy(zeros_hbm, acc)

    plsc.subcore_barrier()

    @pl.loop(0, N_CHUNKS)
    def _(j):
      pltpu.sync_copy(ones_t, acc.at[dst_v.at[j]], add=True)

    plsc.subcore_barrier()
    sl = pl.ds(s * ROWS_PER_TILE, ROWS_PER_TILE)
    pltpu.sync_copy(acc.at[sl], out_hbm.at[c, sl])

  return k(dst_r, ones_v, zeros_v)


def _sc_scatter(h, src_r, dst_r, zeros_v):
  @functools.partial(
      pl.kernel,
      out_type=jax.ShapeDtypeStruct((2, N_PAD, D), jnp.float32),
      mesh=_mesh(),
      scratch_types=[
          pltpu.VMEM((N_CHUNKS, CHUNK), jnp.int32),
          pltpu.VMEM((N_CHUNKS, CHUNK), jnp.int32),
          pltpu.VMEM((CHUNK, D), jnp.float32),
          pltpu.VMEM_SHARED((N_PAD, D), jnp.float32),
      ],
  )
  def k(h_hbm, src_hbm, dst_hbm, zeros_hbm, out_hbm,
        src_v, dst_v, rows_t, acc):
    c = lax.axis_index("c")
    s = lax.axis_index("s")
    wid = c * 16 + s
    pltpu.sync_copy(src_hbm.at[wid], src_v)
    pltpu.sync_copy(dst_hbm.at[wid], dst_v)

    sl = pl.ds(s * ROWS_PER_TILE, ROWS_PER_TILE)
    pltpu.sync_copy(zeros_hbm, acc.at[sl])

    plsc.subcore_barrier()

    @pl.loop(0, N_CHUNKS)
    def _(j):
      pltpu.sync_copy(h_hbm.at[src_v.at[j]], rows_t)
      pltpu.sync_copy(rows_t, acc.at[dst_v.at[j]], add=True)

    plsc.subcore_barrier()
    pltpu.sync_copy(acc.at[sl], out_hbm.at[c, sl])

  return k(h, src_r, dst_r, zeros_v)


def _tc_first(x, d0, d1, W1):
  def body(x_ref, d0_ref, d1_ref, w_ref, h_ref, dinv_ref):
    deg = d0_ref[:, 0:1] + d1_ref[:, 0:1] + 1.0
    dinv = lax.rsqrt(deg)
    dinv_ref[...] = dinv
    h_ref[...] = jnp.dot(x_ref[...] * dinv, w_ref[...],
                         preferred_element_type=jnp.float32)

  return pl.pallas_call(
      body,
      grid=(N_ROW_BLOCKS,),
      in_specs=[
          pl.BlockSpec((ROW_BLOCK, D), lambda i: (i, 0)),
          pl.BlockSpec((ROW_BLOCK, 16), lambda i: (i, 0)),
          pl.BlockSpec((ROW_BLOCK, 16), lambda i: (i, 0)),
          pl.BlockSpec((D, D), lambda i: (0, 0)),
      ],
      out_specs=[
          pl.BlockSpec((ROW_BLOCK, D), lambda i: (i, 0)),
          pl.BlockSpec((ROW_BLOCK, 1), lambda i: (i, 0)),
      ],
      out_shape=[
          jax.ShapeDtypeStruct((N_NODES, D), jnp.float32),
          jax.ShapeDtypeStruct((N_NODES, 1), jnp.float32),
      ],
  )(x, d0, d1, W1)


def _tc_mid(s0, s1, h1, dinv, b1, W2):
  def body(s0_ref, s1_ref, h_ref, dinv_ref, b_ref, w_ref, out_ref):
    dinv = dinv_ref[...]
    t = (s0_ref[...] + s1_ref[...] + h_ref[...]) * dinv + b_ref[...]
    z = jnp.maximum(t, 0.0)
    out_ref[...] = jnp.dot(z * dinv, w_ref[...],
                           preferred_element_type=jnp.float32)

  return pl.pallas_call(
      body,
      grid=(N_ROW_BLOCKS,),
      in_specs=[
          pl.BlockSpec((ROW_BLOCK, D), lambda i: (i, 0)),
          pl.BlockSpec((ROW_BLOCK, D), lambda i: (i, 0)),
          pl.BlockSpec((ROW_BLOCK, D), lambda i: (i, 0)),
          pl.BlockSpec((ROW_BLOCK, 1), lambda i: (i, 0)),
          pl.BlockSpec((D,), lambda i: (0,)),
          pl.BlockSpec((D, D), lambda i: (0, 0)),
      ],
      out_specs=pl.BlockSpec((ROW_BLOCK, D), lambda i: (i, 0)),
      out_shape=jax.ShapeDtypeStruct((N_NODES, D), jnp.float32),
  )(s0, s1, h1, dinv, b1, W2)


def _tc_final(s0, s1, h2, dinv, b2, Wl, bl, batch):
  def body(s0_ref, s1_ref, h_ref, dinv_ref, b_ref, wl_ref, bl_ref, batch_ref,
           out_ref, sums, counts):
    i = pl.program_id(0)

    @pl.when(i == 0)
    def _():
      sums[...] = jnp.zeros_like(sums)
      counts[...] = jnp.zeros_like(counts)

    dinv = dinv_ref[...]
    t = (s0_ref[...] + s1_ref[...] + h_ref[...]) * dinv + b_ref[...]
    z = jnp.maximum(t, 0.0)
    r = jnp.dot(z, wl_ref[...], preferred_element_type=jnp.float32)
    ids = batch_ref[...].reshape(1, ROW_BLOCK)
    onehot = (ids ==
              lax.broadcasted_iota(jnp.int32, (N_GRAPHS, ROW_BLOCK), 0)
              ).astype(jnp.float32)
    sums[...] += jnp.dot(onehot, r, preferred_element_type=jnp.float32)
    counts[...] += jnp.sum(onehot, axis=1, keepdims=True)

    @pl.when(i == N_ROW_BLOCKS - 1)
    def _():
      out_ref[...] = sums[...] / jnp.maximum(counts[...], 1.0) + bl_ref[...]

  return pl.pallas_call(
      body,
      grid=(N_ROW_BLOCKS,),
      in_specs=[
          pl.BlockSpec((ROW_BLOCK, D), lambda i: (i, 0)),
          pl.BlockSpec((ROW_BLOCK, D), lambda i: (i, 0)),
          pl.BlockSpec((ROW_BLOCK, D), lambda i: (i, 0)),
          pl.BlockSpec((ROW_BLOCK, 1), lambda i: (i, 0)),
          pl.BlockSpec((D,), lambda i: (0,)),
          pl.BlockSpec((D, 1), lambda i: (0, 0)),
          pl.BlockSpec((1,), lambda i: (0,)),
          pl.BlockSpec((1, 1, ROW_BLOCK), lambda i: (i, 0, 0)),
      ],
      out_specs=pl.BlockSpec((N_GRAPHS, 1), lambda i: (0, 0)),
      out_shape=jax.ShapeDtypeStruct((N_GRAPHS, 1), jnp.float32),
      scratch_shapes=[
          pltpu.VMEM((N_GRAPHS, 1), jnp.float32),
          pltpu.VMEM((N_GRAPHS, 1), jnp.float32),
      ],
  )(s0, s1, h2, dinv, b2, Wl, bl, batch)


def kernel(x, edge_index, batch, W1, b1, W2, b2, Wl, bl):
  src = edge_index[0].astype(jnp.int32).reshape(N_WORKERS, N_CHUNKS, CHUNK)
  dst = edge_index[1].astype(jnp.int32).reshape(N_WORKERS, N_CHUNKS, CHUNK)
  batch = batch.astype(jnp.int32).reshape(N_ROW_BLOCKS, 1, ROW_BLOCK)

  ones16 = jnp.ones((CHUNK, 16), jnp.float32)
  zeros16 = jnp.zeros((N_PAD, 16), jnp.float32)
  zeros128 = jnp.zeros((ROWS_PER_TILE, D), jnp.float32)

  deg_p = _sc_degree(dst, ones16, zeros16)
  h1, dinv = _tc_first(x, deg_p[0, :N_NODES], deg_p[1, :N_NODES], W1)
  s1 = _sc_scatter(h1, src, dst, zeros128)
  h2 = _tc_mid(s1[0, :N_NODES], s1[1, :N_NODES], h1, dinv, b1, W2)
  s2 = _sc_scatter(h2, src, dst, zeros128)
  return _tc_final(s2[0, :N_NODES], s2[1, :N_NODES], h2, dinv, b2, Wl, bl,
                   batch)


# speedup vs baseline: 20.0148x; 1.0738x over previous
import functools

import jax
import jax.numpy as jnp
from jax import lax
from jax.experimental import pallas as pl
from jax.experimental.pallas import tpu as pltpu
from jax.experimental.pallas import tpu_sc as plsc

N_NODES = 10000
N_EDGES = 320000
D = 128
N_GRAPHS = 64

N_WORKERS = 32
CHUNK = 100
N_CHUNKS = N_EDGES // (N_WORKERS * CHUNK)
N_PAIRS = N_CHUNKS // 2
N_PAD = 10240
ROWS_PER_TILE = N_PAD // 16
ROW_BLOCK = 2000
N_ROW_BLOCKS = N_NODES // ROW_BLOCK

_mesh = functools.partial(
    plsc.VectorSubcoreMesh, core_axis_name="c", subcore_axis_name="s")


def _sc_degree(dst_r, ones_v, zeros_v):
  @functools.partial(
      pl.kernel,
      out_type=jax.ShapeDtypeStruct((2, N_PAD, 16), jnp.float32),
      mesh=_mesh(),
      scratch_types=[
          pltpu.VMEM((N_CHUNKS, CHUNK), jnp.int32),
          pltpu.VMEM((CHUNK, 16), jnp.float32),
          pltpu.VMEM_SHARED((N_PAD, 16), jnp.float32),
      ],
  )
  def k(dst_hbm, ones_hbm, zeros_hbm, out_hbm, dst_v, ones_t, acc):
    c = lax.axis_index("c")
    s = lax.axis_index("s")
    wid = c * 16 + s
    pltpu.sync_copy(dst_hbm.at[wid], dst_v)
    pltpu.sync_copy(ones_hbm, ones_t)

    @pl.when(s == 0)
    def _():
      pltpu.sync_copy(zeros_hbm, acc)

    plsc.subcore_barrier()

    @pl.loop(0, N_CHUNKS)
    def _(j):
      pltpu.sync_copy(ones_t, acc.at[dst_v.at[j]], add=True)

    plsc.subcore_barrier()
    sl = pl.ds(s * ROWS_PER_TILE, ROWS_PER_TILE)
    pltpu.sync_copy(acc.at[sl], out_hbm.at[c, sl])

  return k(dst_r, ones_v, zeros_v)


def _sc_scatter(h, src_r, dst_r, zeros_v):
  @functools.partial(
      pl.kernel,
      out_type=jax.ShapeDtypeStruct((2, N_PAD, D), jnp.float32),
      mesh=_mesh(),
      scratch_types=[
          pltpu.VMEM((N_CHUNKS, CHUNK), jnp.int32),
          pltpu.VMEM((N_CHUNKS, CHUNK), jnp.int32),
          pltpu.VMEM((CHUNK, D), jnp.float32),
          pltpu.VMEM_SHARED((N_PAD, D), jnp.float32),
      ],
  )
  def k(h_hbm, src_hbm, dst_hbm, zeros_hbm, out_hbm,
        src_v, dst_v, rows_t, acc):
    c = lax.axis_index("c")
    s = lax.axis_index("s")
    wid = c * 16 + s
    pltpu.sync_copy(src_hbm.at[wid], src_v)
    pltpu.sync_copy(dst_hbm.at[wid], dst_v)

    sl = pl.ds(s * ROWS_PER_TILE, ROWS_PER_TILE)
    pltpu.sync_copy(zeros_hbm, acc.at[sl])

    plsc.subcore_barrier()

    @pl.loop(0, N_CHUNKS)
    def _(j):
      pltpu.sync_copy(h_hbm.at[src_v.at[j]], rows_t)
      pltpu.sync_copy(rows_t, acc.at[dst_v.at[j]], add=True)

    plsc.subcore_barrier()
    pltpu.sync_copy(acc.at[sl], out_hbm.at[c, sl])

  return k(h, src_r, dst_r, zeros_v)


def _tc_first(x, d0, d1, W1):
  def body(x_ref, d0_ref, d1_ref, w_ref, h_ref, dinv_ref):
    deg = d0_ref[:, 0:1] + d1_ref[:, 0:1] + 1.0
    dinv = lax.rsqrt(deg)
    dinv_ref[...] = dinv
    h_ref[...] = jnp.dot(x_ref[...] * dinv, w_ref[...],
                         preferred_element_type=jnp.float32)

  return pl.pallas_call(
      body,
      grid=(N_ROW_BLOCKS,),
      in_specs=[
          pl.BlockSpec((ROW_BLOCK, D), lambda i: (i, 0)),
          pl.BlockSpec((ROW_BLOCK, 16), lambda i: (i, 0)),
          pl.BlockSpec((ROW_BLOCK, 16), lambda i: (i, 0)),
          pl.BlockSpec((D, D), lambda i: (0, 0)),
      ],
      out_specs=[
          pl.BlockSpec((ROW_BLOCK, D), lambda i: (i, 0)),
          pl.BlockSpec((ROW_BLOCK, 1), lambda i: (i, 0)),
      ],
      out_shape=[
          jax.ShapeDtypeStruct((N_NODES, D), jnp.float32),
          jax.ShapeDtypeStruct((N_NODES, 1), jnp.float32),
      ],
  )(x, d0, d1, W1)


def _tc_mid(s0, s1, h1, dinv, b1, W2):
  def body(s0_ref, s1_ref, h_ref, dinv_ref, b_ref, w_ref, out_ref):
    dinv = dinv_ref[...]
    t = (s0_ref[...] + s1_ref[...] + h_ref[...]) * dinv + b_ref[...]
    z = jnp.maximum(t, 0.0)
    out_ref[...] = jnp.dot(z * dinv, w_ref[...],
                           preferred_element_type=jnp.float32)

  return pl.pallas_call(
      body,
      grid=(N_ROW_BLOCKS,),
      in_specs=[
          pl.BlockSpec((ROW_BLOCK, D), lambda i: (i, 0)),
          pl.BlockSpec((ROW_BLOCK, D), lambda i: (i, 0)),
          pl.BlockSpec((ROW_BLOCK, D), lambda i: (i, 0)),
          pl.BlockSpec((ROW_BLOCK, 1), lambda i: (i, 0)),
          pl.BlockSpec((D,), lambda i: (0,)),
          pl.BlockSpec((D, D), lambda i: (0, 0)),
      ],
      out_specs=pl.BlockSpec((ROW_BLOCK, D), lambda i: (i, 0)),
      out_shape=jax.ShapeDtypeStruct((N_NODES, D), jnp.float32),
  )(s0, s1, h1, dinv, b1, W2)


def _tc_final(s0, s1, h2, dinv, b2, Wl, bl, batch):
  def body(s0_ref, s1_ref, h_ref, dinv_ref, b_ref, wl_ref, bl_ref, batch_ref,
           out_ref, sums, counts):
    i = pl.program_id(0)

    @pl.when(i == 0)
    def _():
      sums[...] = jnp.zeros_like(sums)
      counts[...] = jnp.zeros_like(counts)

    dinv = dinv_ref[...]
    t = (s0_ref[...] + s1_ref[...] + h_ref[...]) * dinv + b_ref[...]
    z = jnp.maximum(t, 0.0)
    r = jnp.dot(z, wl_ref[...], preferred_element_type=jnp.float32)
    ids = batch_ref[...].reshape(1, ROW_BLOCK)
    onehot = (ids ==
              lax.broadcasted_iota(jnp.int32, (N_GRAPHS, ROW_BLOCK), 0)
              ).astype(jnp.float32)
    sums[...] += jnp.dot(onehot, r, preferred_element_type=jnp.float32)
    counts[...] += jnp.sum(onehot, axis=1, keepdims=True)

    @pl.when(i == N_ROW_BLOCKS - 1)
    def _():
      out_ref[...] = sums[...] / jnp.maximum(counts[...], 1.0) + bl_ref[...]

  return pl.pallas_call(
      body,
      grid=(N_ROW_BLOCKS,),
      in_specs=[
          pl.BlockSpec((ROW_BLOCK, D), lambda i: (i, 0)),
          pl.BlockSpec((ROW_BLOCK, D), lambda i: (i, 0)),
          pl.BlockSpec((ROW_BLOCK, D), lambda i: (i, 0)),
          pl.BlockSpec((ROW_BLOCK, 1), lambda i: (i, 0)),
          pl.BlockSpec((D,), lambda i: (0,)),
          pl.BlockSpec((D, 1), lambda i: (0, 0)),
          pl.BlockSpec((1,), lambda i: (0,)),
          pl.BlockSpec((1, 1, ROW_BLOCK), lambda i: (i, 0, 0)),
      ],
      out_specs=pl.BlockSpec((N_GRAPHS, 1), lambda i: (0, 0)),
      out_shape=jax.ShapeDtypeStruct((N_GRAPHS, 1), jnp.float32),
      scratch_shapes=[
          pltpu.VMEM((N_GRAPHS, 1), jnp.float32),
          pltpu.VMEM((N_GRAPHS, 1), jnp.float32),
      ],
  )(s0, s1, h2, dinv, b2, Wl, bl, batch)


def kernel(x, edge_index, batch, W1, b1, W2, b2, Wl, bl):
  src = edge_index[0].astype(jnp.int32).reshape(N_WORKERS, N_CHUNKS, CHUNK)
  dst = edge_index[1].astype(jnp.int32).reshape(N_WORKERS, N_CHUNKS, CHUNK)
  batch = batch.astype(jnp.int32).reshape(N_ROW_BLOCKS, 1, ROW_BLOCK)

  ones16 = jnp.ones((CHUNK, 16), jnp.float32)
  zeros16 = jnp.zeros((N_PAD, 16), jnp.float32)
  zeros128 = jnp.zeros((ROWS_PER_TILE, D), jnp.float32)

  deg_p = _sc_degree(dst, ones16, zeros16)
  h1, dinv = _tc_first(x, deg_p[0, :N_NODES], deg_p[1, :N_NODES], W1)
  s1 = _sc_scatter(h1, src, dst, zeros128)
  h2 = _tc_mid(s1[0, :N_NODES], s1[1, :N_NODES], h1, dinv, b1, W2)
  s2 = _sc_scatter(h2, src, dst, zeros128)
  return _tc_final(s2[0, :N_NODES], s2[1, :N_NODES], h2, dinv, b2, Wl, bl,
                   batch)
